# baseline (speedup 1.0000x reference)
import jax
import jax.numpy as jnp
from jax import lax
from jax.experimental import pallas as pl
from jax.experimental.pallas import tpu as pltpu, tpu_sc as plsc

VOCAB = 1000000
EMBED_DIM = 64
BATCH = 4096
HIST = 200

NC = 2
NS = 16
NW = NC * NS

TH = HIST // 8
TB = BATCH // 128
HL = 8
BL = 128
TE = EMBED_DIM // 8
R = 8
L = 128

NCOL = VOCAB // 128
VTAIL = NCOL * 128
COLS_PER_W = 245


def _relayout_body(tab_hbm, tail_hbm, out_hbm, col0, col1, ob0, ob1,
                   gsem0, gsem1, wsem0, wsem1):
  w = lax.axis_index("s") * NC + lax.axis_index("c")
  base = w * COLS_PER_W

  lanes = lax.iota(jnp.int32, 16)

  def col_of(i):
    return jnp.minimum(base + i, NCOL - 1)

  def fire_in(i, buf, sem):
    pltpu.async_copy(tab_hbm.at[:, pl.ds(col_of(i) * 128, 128)], buf, sem)

  def drain_in(buf, sem):
    pltpu.make_async_copy(tab_hbm.at[:, pl.ds(0, 128)], buf, sem).wait()

  half = lax.bitwise_and(lanes, 1) * 64

  def transpose_col(src, dst):
    @plsc.parallel_loop(0, 16, unroll=4)
    def kstep(k):
      perm = lax.bitwise_and(lanes + k, 15)
      cperm = perm + half
      for b0 in range(0, 128, 16):
        rvec = lax.shift_right_logical(b0 + lanes, 1)
        for e0 in range(0, EMBED_DIM, 16):
          vals = plsc.load_gather(src, [e0 + perm, b0 + lanes])
          plsc.store_scatter(dst, [rvec, e0 + cperm], vals)

  def fire_out(i, buf, sem):
    pltpu.async_copy(buf, out_hbm.at[pl.ds(col_of(i) * 64, 64)], sem)

  def drain_out(buf, sem):
    pltpu.make_async_copy(buf, out_hbm.at[pl.ds(0, 64)], sem).wait()

  fire_in(0, col0, gsem0)

  def pair_body(i, carry):
    i0 = 2 * i
    drain_in(col0, gsem0)

    @pl.when(i > 0)
    def _():
      drain_out(ob0, wsem0)

    fire_in(i0 + 1, col1, gsem1)
    transpose_col(col0, ob0)
    fire_out(i0, ob0, wsem0)
    drain_in(col1, gsem1)

    @pl.when(i < COLS_PER_W // 2 - 1)
    def _():
      fire_in(i0 + 2, col0, gsem0)

    @pl.when(i > 0)
    def _():
      drain_out(ob1, wsem1)

    transpose_col(col1, ob1)
    fire_out(i0 + 1, ob1, wsem1)
    return carry

  lax.fori_loop(0, COLS_PER_W // 2, pair_body, 0)
  drain_out(ob0, wsem0)
  fire_in(COLS_PER_W - 1, col0, gsem0)
  drain_in(col0, gsem0)
  transpose_col(col0, ob0)
  fire_out(COLS_PER_W - 1, ob0, wsem0)
  drain_out(ob0, wsem0)
  drain_out(ob1, wsem1)

  @pl.when(w == NW - 1)
  def _():
    pltpu.sync_copy(tail_hbm, out_hbm.at[pl.ds(VTAIL // 2, 32)])


@jax.jit
def _relayout(tab_t, tail128):
  mesh = plsc.VectorSubcoreMesh(
      core_axis_name="c", subcore_axis_name="s", num_cores=NC, num_subcores=NS
  )
  f = pl.kernel(
      _relayout_body,
      out_type=jax.ShapeDtypeStruct((VOCAB // 2, 128), jnp.float32),
      mesh=mesh,
      scratch_types=[
          pltpu.VMEM((EMBED_DIM, 128), jnp.float32),
          pltpu.VMEM((EMBED_DIM, 128), jnp.float32),
          pltpu.VMEM((EMBED_DIM, 128), jnp.float32),
          pltpu.VMEM((EMBED_DIM, 128), jnp.float32),
          pltpu.SemaphoreType.DMA,
          pltpu.SemaphoreType.DMA,
          pltpu.SemaphoreType.DMA,
          pltpu.SemaphoreType.DMA,
      ],
      compiler_params=pltpu.CompilerParams(
          use_tc_tiling_on_sc=True, needs_layout_passes=False),
  )
  return f(tab_t, tail128)


def _emb_body(idx_hbm, table_hbm, out_hbm, idx_v, rows0, rows1, tb0, tb1,
              gsem0, gsem1, wsem0, wsem1):
  w = lax.axis_index("s") * NC + lax.axis_index("c")

  for th in range(TH):
    pltpu.async_copy(idx_hbm.at[th, w], idx_v.at[th], gsem0)
  for th in range(TH):
    pltpu.make_async_copy(idx_hbm.at[th, w], idx_v.at[th], gsem0).wait()

  lanes = lax.iota(jnp.int32, 16)

  def fire_gather(h, rows, sem):
    pltpu.async_copy(table_hbm.at[idx_v.at[h // HL, h % HL]], rows, sem)

  def drain_gather(rows, sem):
    pltpu.make_async_copy(table_hbm.at[idx_v.at[0, 0]], rows, sem).wait()

  def transpose_unit(rows, tbuf):
    @plsc.parallel_loop(0, 16, unroll=4)
    def kstep(k):
      perm = lax.bitwise_and(lanes + k, 15)
      for b0 in range(0, BL, 16):
        for e0 in range(0, EMBED_DIM, 16):
          vals = plsc.load_gather(rows, [b0 + lanes, e0 + perm])
          plsc.store_scatter(tbuf, [e0 + perm, b0 + lanes], vals)

  def fire_writes(h, tbuf, sem):
    for te in range(TE):
      pltpu.async_copy(tbuf.at[pl.ds(te * R, R)], out_hbm.at[h, te, w], sem)

  def drain_writes(h, tbuf, sem):
    for te in range(TE):
      pltpu.make_async_copy(
          tbuf.at[pl.ds(te * R, R)], out_hbm.at[h, te, w], sem).wait()

  fire_gather(0, rows0, gsem0)

  def pair_body(i, carry):
    h0 = 2 * i
    drain_gather(rows0, gsem0)

    @pl.when(i > 0)
    def _():
      drain_writes(h0, tb0, wsem0)

    fire_gather(h0 + 1, rows1, gsem1)
    transpose_unit(rows0, tb0)
    fire_writes(h0, tb0, wsem0)
    drain_gather(rows1, gsem1)

    @pl.when(i < HIST // 2 - 1)
    def _():
      fire_gather(h0 + 2, rows0, gsem0)

    @pl.when(i > 0)
    def _():
      drain_writes(h0, tb1, wsem1)

    transpose_unit(rows1, tb1)
    fire_writes(h0 + 1, tb1, wsem1)
    return carry

  lax.fori_loop(0, HIST // 2, pair_body, 0)
  drain_writes(0, tb0, wsem0)
  drain_writes(0, tb1, wsem1)


@jax.jit
def _emb_lookup(idx, table):
  mesh = plsc.VectorSubcoreMesh(
      core_axis_name="c", subcore_axis_name="s", num_cores=NC, num_subcores=NS
  )
  f = pl.kernel(
      _emb_body,
      out_type=jax.ShapeDtypeStruct((HIST, TE, TB, R, L), jnp.float32),
      mesh=mesh,
      scratch_types=[
          pltpu.VMEM((TH, HL, BL), jnp.int32),
          pltpu.VMEM((BL, EMBED_DIM), jnp.float32),
          pltpu.VMEM((BL, EMBED_DIM), jnp.float32),
          pltpu.VMEM((EMBED_DIM, BL), jnp.float32),
          pltpu.VMEM((EMBED_DIM, BL), jnp.float32),
          pltpu.SemaphoreType.DMA,
          pltpu.SemaphoreType.DMA,
          pltpu.SemaphoreType.DMA,
          pltpu.SemaphoreType.DMA,
      ],
      compiler_params=pltpu.CompilerParams(
          use_tc_tiling_on_sc=False, needs_layout_passes=False),
  )
  return f(idx, table)


def kernel(input, embed_vecs):
  idx = input.astype(jnp.int32).reshape(TB, BL, TH, HL).transpose(2, 0, 3, 1)
  tab_t = embed_vecs.T
  tail = embed_vecs[VTAIL:].reshape(32, 128)
  scratch = _relayout(tab_t, tail)
  out5 = _emb_lookup(idx, scratch.reshape(VOCAB, EMBED_DIM))
  return out5.transpose(2, 4, 0, 1, 3).reshape(BATCH, HIST, EMBED_DIM)


# speedup vs baseline: 1.0884x; 1.0884x over previous
import jax
import jax.numpy as jnp
from jax import lax
from jax.experimental import pallas as pl
from jax.experimental.pallas import tpu as pltpu, tpu_sc as plsc

VOCAB = 1000000
EMBED_DIM = 64
BATCH = 4096
HIST = 200

NC = 2
NS = 16
NW = NC * NS

TH = HIST // 8
TB = BATCH // 128
HL = 8
BL = 128
TE = EMBED_DIM // 8
R = 8
L = 128

NCOL = VOCAB // 128
VTAIL = NCOL * 128
COLS_PER_W = 245


def _relayout_body(tab_hbm, tail_hbm, out_hbm, col0, col1, ob0, ob1,
                   gsem0, gsem1, wsem0, wsem1):
  w = lax.axis_index("s") * NC + lax.axis_index("c")
  base = w * COLS_PER_W

  lanes = lax.iota(jnp.int32, 16)

  def col_of(i):
    return jnp.minimum(base + i, NCOL - 1)

  def fire_in(i, buf, sem):
    pltpu.async_copy(tab_hbm.at[:, pl.ds(col_of(i) * 128, 128)], buf, sem)

  def drain_in(buf, sem):
    pltpu.make_async_copy(tab_hbm.at[:, pl.ds(0, 128)], buf, sem).wait()

  half = lax.bitwise_and(lanes, 1) * 64

  def transpose_col(src, dst):
    @plsc.parallel_loop(0, 16)
    def kstep(k):
      perm = lax.bitwise_and(lanes + k, 15)
      cperm = perm + half
      for b0 in range(0, 128, 16):
        rvec = lax.shift_right_logical(b0 + lanes, 1)
        for e0 in range(0, EMBED_DIM, 16):
          vals = plsc.load_gather(src, [e0 + perm, b0 + lanes])
          plsc.store_scatter(dst, [rvec, e0 + cperm], vals)

  def fire_out(i, buf, sem):
    pltpu.async_copy(buf, out_hbm.at[pl.ds(col_of(i) * 64, 64)], sem)

  def drain_out(buf, sem):
    pltpu.make_async_copy(buf, out_hbm.at[pl.ds(0, 64)], sem).wait()

  fire_in(0, col0, gsem0)

  def pair_body(i, carry):
    i0 = 2 * i
    drain_in(col0, gsem0)

    @pl.when(i > 0)
    def _():
      drain_out(ob0, wsem0)

    fire_in(i0 + 1, col1, gsem1)
    transpose_col(col0, ob0)
    fire_out(i0, ob0, wsem0)
    drain_in(col1, gsem1)

    @pl.when(i < COLS_PER_W // 2 - 1)
    def _():
      fire_in(i0 + 2, col0, gsem0)

    @pl.when(i > 0)
    def _():
      drain_out(ob1, wsem1)

    transpose_col(col1, ob1)
    fire_out(i0 + 1, ob1, wsem1)
    return carry

  lax.fori_loop(0, COLS_PER_W // 2, pair_body, 0)
  drain_out(ob0, wsem0)
  fire_in(COLS_PER_W - 1, col0, gsem0)
  drain_in(col0, gsem0)
  transpose_col(col0, ob0)
  fire_out(COLS_PER_W - 1, ob0, wsem0)
  drain_out(ob0, wsem0)
  drain_out(ob1, wsem1)

  @pl.when(w == NW - 1)
  def _():
    pltpu.sync_copy(tail_hbm, out_hbm.at[pl.ds(VTAIL // 2, 32)])


@jax.jit
def _relayout(tab_t, tail128):
  mesh = plsc.VectorSubcoreMesh(
      core_axis_name="c", subcore_axis_name="s", num_cores=NC, num_subcores=NS
  )
  f = pl.kernel(
      _relayout_body,
      out_type=jax.ShapeDtypeStruct((VOCAB // 2, 128), jnp.float32),
      mesh=mesh,
      scratch_types=[
          pltpu.VMEM((EMBED_DIM, 128), jnp.float32),
          pltpu.VMEM((EMBED_DIM, 128), jnp.float32),
          pltpu.VMEM((EMBED_DIM, 128), jnp.float32),
          pltpu.VMEM((EMBED_DIM, 128), jnp.float32),
          pltpu.SemaphoreType.DMA,
          pltpu.SemaphoreType.DMA,
          pltpu.SemaphoreType.DMA,
          pltpu.SemaphoreType.DMA,
      ],
      compiler_params=pltpu.CompilerParams(
          use_tc_tiling_on_sc=True, needs_layout_passes=False),
  )
  return f(tab_t, tail128)


def _emb_body(idx_hbm, table_hbm, out_hbm, idx_v, rows0, rows1, tb0, tb1,
              gsem0, gsem1, wsem0, wsem1):
  w = lax.axis_index("s") * NC + lax.axis_index("c")

  for th in range(TH):
    pltpu.async_copy(idx_hbm.at[th, w], idx_v.at[th], gsem0)
  for th in range(TH):
    pltpu.make_async_copy(idx_hbm.at[th, w], idx_v.at[th], gsem0).wait()

  lanes = lax.iota(jnp.int32, 16)

  def fire_gather(h, rows, sem):
    pltpu.async_copy(table_hbm.at[idx_v.at[h // HL, h % HL]], rows, sem)

  def drain_gather(rows, sem):
    pltpu.make_async_copy(table_hbm.at[idx_v.at[0, 0]], rows, sem).wait()

  def transpose_unit(rows, tbuf):
    @plsc.parallel_loop(0, 16)
    def kstep(k):
      perm = lax.bitwise_and(lanes + k, 15)
      for b0 in range(0, BL, 16):
        for e0 in range(0, EMBED_DIM, 16):
          vals = plsc.load_gather(rows, [b0 + lanes, e0 + perm])
          plsc.store_scatter(tbuf, [e0 + perm, b0 + lanes], vals)

  def fire_writes(h, tbuf, sem):
    for te in range(TE):
      pltpu.async_copy(tbuf.at[pl.ds(te * R, R)], out_hbm.at[h, te, w], sem)

  def drain_writes(h, tbuf, sem):
    for te in range(TE):
      pltpu.make_async_copy(
          tbuf.at[pl.ds(te * R, R)], out_hbm.at[h, te, w], sem).wait()

  fire_gather(0, rows0, gsem0)

  def pair_body(i, carry):
    h0 = 2 * i
    drain_gather(rows0, gsem0)

    @pl.when(i > 0)
    def _():
      drain_writes(h0, tb0, wsem0)

    fire_gather(h0 + 1, rows1, gsem1)
    transpose_unit(rows0, tb0)
    fire_writes(h0, tb0, wsem0)
    drain_gather(rows1, gsem1)

    @pl.when(i < HIST // 2 - 1)
    def _():
      fire_gather(h0 + 2, rows0, gsem0)

    @pl.when(i > 0)
    def _():
      drain_writes(h0, tb1, wsem1)

    transpose_unit(rows1, tb1)
    fire_writes(h0 + 1, tb1, wsem1)
    return carry

  lax.fori_loop(0, HIST // 2, pair_body, 0)
  drain_writes(0, tb0, wsem0)
  drain_writes(0, tb1, wsem1)


@jax.jit
def _emb_lookup(idx, table):
  mesh = plsc.VectorSubcoreMesh(
      core_axis_name="c", subcore_axis_name="s", num_cores=NC, num_subcores=NS
  )
  f = pl.kernel(
      _emb_body,
      out_type=jax.ShapeDtypeStruct((HIST, TE, TB, R, L), jnp.float32),
      mesh=mesh,
      scratch_types=[
          pltpu.VMEM((TH, HL, BL), jnp.int32),
          pltpu.VMEM((BL, EMBED_DIM), jnp.float32),
          pltpu.VMEM((BL, EMBED_DIM), jnp.float32),
          pltpu.VMEM((EMBED_DIM, BL), jnp.float32),
          pltpu.VMEM((EMBED_DIM, BL), jnp.float32),
          pltpu.SemaphoreType.DMA,
          pltpu.SemaphoreType.DMA,
          pltpu.SemaphoreType.DMA,
          pltpu.SemaphoreType.DMA,
      ],
      compiler_params=pltpu.CompilerParams(
          use_tc_tiling_on_sc=False, needs_layout_passes=False),
  )
  return f(idx, table)


def kernel(input, embed_vecs):
  idx = input.astype(jnp.int32).reshape(TB, BL, TH, HL).transpose(2, 0, 3, 1)
  tab_t = embed_vecs.T
  tail = embed_vecs[VTAIL:].reshape(32, 128)
  scratch = _relayout(tab_t, tail)
  out5 = _emb_lookup(idx, scratch.reshape(VOCAB, EMBED_DIM))
  return out5.transpose(2, 4, 0, 1, 3).reshape(BATCH, HIST, EMBED_DIM)
